# baseline (speedup 1.0000x reference)
import jax
import jax.numpy as jnp
from jax import lax
from jax.experimental import pallas as pl
from jax.experimental.pallas import tpu as pltpu
from jax.experimental.pallas import tpu_sc as plsc

_NC = 2
_NS = 16
_CH = 128
_G = 64


def _edge_agg_sc(y, src_t, dst_t, z_feat, z_cnt, ones_blk, nrows, n_chunks,
                 k0, with_cnt):
  w = y.shape[1]
  rpt = nrows // _NS
  mesh = plsc.VectorSubcoreMesh(core_axis_name="c", subcore_axis_name="s",
                                num_cores=_NC, num_subcores=_NS)
  outs = [jax.ShapeDtypeStruct((_NC, nrows, w), jnp.float32)]
  scratch = [
      pltpu.VMEM_SHARED((nrows, w), jnp.float32),
      pltpu.VMEM((n_chunks, _CH), jnp.int32),
      pltpu.VMEM((n_chunks, _CH), jnp.int32),
      pltpu.VMEM((_CH, w), jnp.float32),
      pltpu.SemaphoreType.DMA,
  ]
  if with_cnt:
    outs.append(jax.ShapeDtypeStruct((_NC, nrows, 16), jnp.float32))
    scratch += [
        pltpu.VMEM_SHARED((nrows, 16), jnp.float32),
        pltpu.VMEM((_CH, 16), jnp.float32),
    ]

  def body(*refs):
    if with_cnt:
      (y_h, src_h, dst_h, zf_h, zc_h, ones_h, agg_o, cnt_o,
       agg_sh, src_v, dst_v, rows_a, sem_a, cnt_sh, ones_v) = refs
    else:
      (y_h, src_h, dst_h, zf_h, agg_o,
       agg_sh, src_v, dst_v, rows_a, sem_a) = refs
    c = lax.axis_index("c")
    s = lax.axis_index("s")
    r0 = s * rpt
    pltpu.sync_copy(zf_h.at[pl.ds(r0, rpt)], agg_sh.at[pl.ds(r0, rpt)])
    if with_cnt:
      pltpu.sync_copy(zc_h.at[pl.ds(r0, rpt)], cnt_sh.at[pl.ds(r0, rpt)])
      pltpu.sync_copy(ones_h, ones_v)
    pltpu.sync_copy(src_h.at[c, s], src_v)
    pltpu.sync_copy(dst_h.at[c, s], dst_v)
    plsc.subcore_barrier()

    myk = jnp.where(c == 0, k0, n_chunks)

    @pl.loop(0, myk)
    def _(j):
      pltpu.async_copy(y_h.at[src_v.at[j]], rows_a, sem_a).wait()
      pltpu.sync_copy(rows_a, agg_sh.at[dst_v.at[j]], add=True)
      if with_cnt:
        pltpu.sync_copy(ones_v, cnt_sh.at[dst_v.at[j]], add=True)

    plsc.subcore_barrier()
    pltpu.sync_copy(agg_sh.at[pl.ds(r0, rpt)], agg_o.at[c, pl.ds(r0, rpt)])
    if with_cnt:
      pltpu.sync_copy(cnt_sh.at[pl.ds(r0, rpt)], cnt_o.at[c, pl.ds(r0, rpt)])

  kern = pl.kernel(
      body, out_type=tuple(outs), mesh=mesh, scratch_types=scratch,
      compiler_params=pltpu.CompilerParams(use_tc_tiling_on_sc=False))
  if with_cnt:
    return kern(y, src_t, dst_t, z_feat, z_cnt, ones_blk)
  return kern(y, src_t, dst_t, z_feat)[0]


def kernel(x, edge_index, batch, Wl1, bl1, Wr1, Wl2, bl2, Wr2, W3, b3, W4,
           b4):
  n, d = x.shape
  h = Wl1.shape[0]
  e = edge_index.shape[1]
  f32 = jnp.float32
  bf16 = jnp.bfloat16

  tiles = _NC * _NS
  total_chunks = -(-e // (_CH * _NS))
  k0 = int(round(total_chunks * 0.48))
  n_chunks = total_chunks - k0
  e_pad = _NS * total_chunks * _CH
  nrows = -(-(n + 1) // (_NS * 8)) * (_NS * 8)

  src = edge_index[0]
  dst = edge_index[1]
  pad = e_pad - e
  def _split(a):
    a = a.reshape(_NS, total_chunks, _CH)
    a0 = jnp.pad(a[:, :k0], ((0, 0), (0, n_chunks - k0), (0, 0)))
    return jnp.stack([a0, a[:, k0:]])
  src_t = _split(jnp.concatenate([src, jnp.zeros((pad,), jnp.int32)]))
  dst_t = _split(jnp.concatenate([dst, jnp.full((pad,), n, jnp.int32)]))
  z_feat = jnp.zeros((nrows, h), f32)
  z_cnt = jnp.zeros((nrows, 16), f32)
  ones_blk = jnp.ones((_CH, 16), f32)

  w1cat = jnp.concatenate([Wl1.T, Wr1.T], axis=1)

  def _mm1(x_ref, w_ref, y_ref, r_ref):
    o = jnp.dot(x_ref[...].astype(bf16), w_ref[...].astype(bf16),
                preferred_element_type=f32)
    y_ref[...] = o[:, :h]
    r_ref[...] = o[:, h:]

  y1, r1 = pl.pallas_call(
      _mm1,
      out_shape=(jax.ShapeDtypeStruct((n, h), f32),
                 jax.ShapeDtypeStruct((n, h), f32)),
  )(x, w1cat)

  agg1, cntp = _edge_agg_sc(y1, src_t, dst_t, z_feat, z_cnt, ones_blk,
                            nrows, n_chunks, k0, True)

  w2cat = jnp.concatenate([Wl2.T, Wr2.T], axis=1)
  bl1r = bl1.reshape(1, h)

  def _mid(a0_ref, a1_ref, c0_ref, c1_ref, r_ref, b_ref, w_ref,
           y_ref, r2_ref, cnt_ref):
    cnt_v = jnp.maximum(c0_ref[...] + c1_ref[...], 1.0)
    h1 = jax.nn.relu((a0_ref[...] + a1_ref[...]) / cnt_v + b_ref[...]
                     + r_ref[...])
    o = jnp.dot(h1.astype(bf16), w_ref[...].astype(bf16),
                preferred_element_type=f32)
    y_ref[...] = o[:, :h]
    r2_ref[...] = o[:, h:]
    cnt_ref[...] = cnt_v

  y2, r2, cnt = pl.pallas_call(
      _mid,
      out_shape=(jax.ShapeDtypeStruct((n, h), f32),
                 jax.ShapeDtypeStruct((n, h), f32),
                 jax.ShapeDtypeStruct((n, 1), f32)),
  )(agg1[0, :n], agg1[1, :n], cntp[0, :n, :1], cntp[1, :n, :1], r1, bl1r,
    w2cat)

  agg2 = _edge_agg_sc(y2, src_t, dst_t, z_feat, None, None,
                      nrows, n_chunks, k0, False)

  bl2r = bl2.reshape(1, h)
  batch_row = batch.reshape(1, n)
  w3t = W3.T
  b3r = b3.reshape(1, -1)
  w4t = W4.T
  b4r = b4.reshape(1, -1)

  def _head(a0_ref, a1_ref, cnt_ref, r_ref, b_ref, bat_ref,
            w3_ref, b3_ref, w4_ref, b4_ref, out_ref):
    h2 = jax.nn.relu((a0_ref[...] + a1_ref[...]) / cnt_ref[...] + b_ref[...]
                     + r_ref[...])
    onehot_t = (lax.broadcasted_iota(jnp.int32, (_G, n), 0)
                == bat_ref[...]).astype(bf16)
    pooled_sum = jnp.dot(onehot_t, h2.astype(bf16),
                         preferred_element_type=f32)
    counts = jnp.dot(onehot_t, jnp.ones((n, 1), bf16),
                     preferred_element_type=f32)
    pooled = pooled_sum / jnp.maximum(counts, 1.0)
    z = jax.nn.relu(jnp.dot(pooled.astype(bf16), w3_ref[...].astype(bf16),
                            preferred_element_type=f32) + b3_ref[...])
    out_ref[...] = jnp.dot(z.astype(bf16), w4_ref[...].astype(bf16),
                           preferred_element_type=f32) + b4_ref[...]

  out = pl.pallas_call(
      _head,
      out_shape=jax.ShapeDtypeStruct((_G, W4.shape[0]), f32),
  )(agg2[0, :n], agg2[1, :n], cnt, r2, bl2r, batch_row, w3t, b3r, w4t, b4r)
  return out


# speedup vs baseline: 1.0108x; 1.0108x over previous
import jax
import jax.numpy as jnp
from jax import lax
from jax.experimental import pallas as pl
from jax.experimental.pallas import tpu as pltpu
from jax.experimental.pallas import tpu_sc as plsc

_NC = 2
_NS = 16
_CH = 128
_G = 64


def _edge_agg_sc(y, src_t, dst_t, z_feat, z_cnt, ones_blk, nrows, n_chunks,
                 k0, with_cnt):
  w = y.shape[1]
  rpt = nrows // _NS
  mesh = plsc.VectorSubcoreMesh(core_axis_name="c", subcore_axis_name="s",
                                num_cores=_NC, num_subcores=_NS)
  outs = [jax.ShapeDtypeStruct((_NC, nrows, w), jnp.float32)]
  scratch = [
      pltpu.VMEM_SHARED((nrows, w), jnp.float32),
      pltpu.VMEM((n_chunks, _CH), jnp.int32),
      pltpu.VMEM((n_chunks, _CH), jnp.int32),
      pltpu.VMEM((_CH, w), jnp.float32),
      pltpu.SemaphoreType.DMA,
  ]
  if with_cnt:
    outs.append(jax.ShapeDtypeStruct((_NC, nrows, 16), jnp.float32))
    scratch += [
        pltpu.VMEM_SHARED((nrows, 16), jnp.float32),
        pltpu.VMEM((_CH, 16), jnp.float32),
    ]

  def body(*refs):
    if with_cnt:
      (y_h, src_h, dst_h, zf_h, zc_h, ones_h, agg_o, cnt_o,
       agg_sh, src_v, dst_v, rows_a, sem_a, cnt_sh, ones_v) = refs
    else:
      (y_h, src_h, dst_h, zf_h, agg_o,
       agg_sh, src_v, dst_v, rows_a, sem_a) = refs
    c = lax.axis_index("c")
    s = lax.axis_index("s")
    r0 = s * rpt
    pltpu.sync_copy(zf_h.at[pl.ds(r0, rpt)], agg_sh.at[pl.ds(r0, rpt)])
    if with_cnt:
      pltpu.sync_copy(zc_h.at[pl.ds(r0, rpt)], cnt_sh.at[pl.ds(r0, rpt)])
      pltpu.sync_copy(ones_h, ones_v)
    pltpu.sync_copy(src_h.at[c, s], src_v)
    pltpu.sync_copy(dst_h.at[c, s], dst_v)
    plsc.subcore_barrier()

    myk = jnp.where(c == 0, k0, n_chunks)

    @pl.loop(0, myk)
    def _(j):
      pltpu.async_copy(y_h.at[src_v.at[j]], rows_a, sem_a).wait()
      pltpu.sync_copy(rows_a, agg_sh.at[dst_v.at[j]], add=True)
      if with_cnt:
        pltpu.sync_copy(ones_v, cnt_sh.at[dst_v.at[j]], add=True)

    plsc.subcore_barrier()
    pltpu.sync_copy(agg_sh.at[pl.ds(r0, rpt)], agg_o.at[c, pl.ds(r0, rpt)])
    if with_cnt:
      pltpu.sync_copy(cnt_sh.at[pl.ds(r0, rpt)], cnt_o.at[c, pl.ds(r0, rpt)])

  kern = pl.kernel(
      body, out_type=tuple(outs), mesh=mesh, scratch_types=scratch,
      compiler_params=pltpu.CompilerParams(use_tc_tiling_on_sc=False))
  if with_cnt:
    return kern(y, src_t, dst_t, z_feat, z_cnt, ones_blk)
  return kern(y, src_t, dst_t, z_feat)[0]


def kernel(x, edge_index, batch, Wl1, bl1, Wr1, Wl2, bl2, Wr2, W3, b3, W4,
           b4):
  n, d = x.shape
  h = Wl1.shape[0]
  e = edge_index.shape[1]
  f32 = jnp.float32
  bf16 = jnp.bfloat16

  tiles = _NC * _NS
  total_chunks = -(-e // (_CH * _NS))
  k0 = int(round(total_chunks * 0.47))
  n_chunks = total_chunks - k0
  e_pad = _NS * total_chunks * _CH
  nrows = -(-(n + 1) // (_NS * 8)) * (_NS * 8)

  src = edge_index[0]
  dst = edge_index[1]
  pad = e_pad - e
  def _split(a):
    a = a.reshape(_NS, total_chunks, _CH)
    a0 = jnp.pad(a[:, :k0], ((0, 0), (0, n_chunks - k0), (0, 0)))
    return jnp.stack([a0, a[:, k0:]])
  src_t = _split(jnp.concatenate([src, jnp.zeros((pad,), jnp.int32)]))
  dst_t = _split(jnp.concatenate([dst, jnp.full((pad,), n, jnp.int32)]))
  z_feat = jnp.zeros((nrows, h), f32)
  z_cnt = jnp.zeros((nrows, 16), f32)
  ones_blk = jnp.ones((_CH, 16), f32)

  w1cat = jnp.concatenate([Wl1.T, Wr1.T], axis=1)

  def _mm1(x_ref, w_ref, y_ref, r_ref):
    o = jnp.dot(x_ref[...].astype(bf16), w_ref[...].astype(bf16),
                preferred_element_type=f32)
    y_ref[...] = o[:, :h]
    r_ref[...] = o[:, h:]

  y1, r1 = pl.pallas_call(
      _mm1,
      out_shape=(jax.ShapeDtypeStruct((n, h), f32),
                 jax.ShapeDtypeStruct((n, h), f32)),
  )(x, w1cat)

  agg1, cntp = _edge_agg_sc(y1, src_t, dst_t, z_feat, z_cnt, ones_blk,
                            nrows, n_chunks, k0, True)

  w2cat = jnp.concatenate([Wl2.T, Wr2.T], axis=1)
  bl1r = bl1.reshape(1, h)

  def _mid(a0_ref, a1_ref, c0_ref, c1_ref, r_ref, b_ref, w_ref,
           y_ref, r2_ref, cnt_ref):
    cnt_v = jnp.maximum(c0_ref[...] + c1_ref[...], 1.0)
    h1 = jax.nn.relu((a0_ref[...] + a1_ref[...]) / cnt_v + b_ref[...]
                     + r_ref[...])
    o = jnp.dot(h1.astype(bf16), w_ref[...].astype(bf16),
                preferred_element_type=f32)
    y_ref[...] = o[:, :h]
    r2_ref[...] = o[:, h:]
    cnt_ref[...] = cnt_v

  y2, r2, cnt = pl.pallas_call(
      _mid,
      out_shape=(jax.ShapeDtypeStruct((n, h), f32),
                 jax.ShapeDtypeStruct((n, h), f32),
                 jax.ShapeDtypeStruct((n, 1), f32)),
  )(agg1[0, :n], agg1[1, :n], cntp[0, :n, :1], cntp[1, :n, :1], r1, bl1r,
    w2cat)

  agg2 = _edge_agg_sc(y2, src_t, dst_t, z_feat, None, None,
                      nrows, n_chunks, k0, False)

  bl2r = bl2.reshape(1, h)
  batch_row = batch.reshape(1, n)
  w3t = W3.T
  b3r = b3.reshape(1, -1)
  w4t = W4.T
  b4r = b4.reshape(1, -1)

  def _head(a0_ref, a1_ref, cnt_ref, r_ref, b_ref, bat_ref,
            w3_ref, b3_ref, w4_ref, b4_ref, out_ref):
    h2 = jax.nn.relu((a0_ref[...] + a1_ref[...]) / cnt_ref[...] + b_ref[...]
                     + r_ref[...])
    onehot_t = (lax.broadcasted_iota(jnp.int32, (_G, n), 0)
                == bat_ref[...]).astype(bf16)
    pooled_sum = jnp.dot(onehot_t, h2.astype(bf16),
                         preferred_element_type=f32)
    counts = jnp.dot(onehot_t, jnp.ones((n, 1), bf16),
                     preferred_element_type=f32)
    pooled = pooled_sum / jnp.maximum(counts, 1.0)
    z = jax.nn.relu(jnp.dot(pooled.astype(bf16), w3_ref[...].astype(bf16),
                            preferred_element_type=f32) + b3_ref[...])
    out_ref[...] = jnp.dot(z.astype(bf16), w4_ref[...].astype(bf16),
                           preferred_element_type=f32) + b4_ref[...]

  out = pl.pallas_call(
      _head,
      out_shape=jax.ShapeDtypeStruct((_G, W4.shape[0]), f32),
  )(agg2[0, :n], agg2[1, :n], cnt, r2, bl2r, batch_row, w3t, b3r, w4t, b4r)
  return out
